# Initial kernel scaffold; baseline (speedup 1.0000x reference)
import functools

import jax
import jax.numpy as jnp
from jax import lax
from jax.experimental import pallas as pl
from jax.experimental.pallas import tpu as pltpu
from jax.experimental.pallas import tpu_sc as plsc

N = 10000
E = 160000
NT = 16
NC = 2
EPT = E // NT
NB = EPT // 16
RPT = N // NT
ZR = 125


@functools.lru_cache(maxsize=None)
def _phase_a(din, cdim, fdim, act):
    nch = cdim // fdim
    bm = 400

    def body(x_ref, w_ref, a2_ref, out_ref, a_ref):
        xb = x_ref[...]
        if act == "elu":
            xb = jnp.where(xb > 0, xb, jnp.expm1(xb))
        prod = jnp.dot(xb, w_ref[...], preferred_element_type=jnp.float32)
        for c in range(nch):
            out_ref[c] = prod[:, c * fdim:(c + 1) * fdim]
        a_ref[...] = jnp.dot(prod, a2_ref[...],
                             preferred_element_type=jnp.float32)

    return pl.pallas_call(
        body,
        grid=(N // bm,),
        in_specs=[
            pl.BlockSpec((bm, din), lambda i: (i, 0)),
            pl.BlockSpec((din, cdim), lambda i: (0, 0)),
            pl.BlockSpec((cdim, 2), lambda i: (0, 0)),
        ],
        out_specs=[
            pl.BlockSpec((nch, bm, fdim), lambda i: (0, i, 0)),
            pl.BlockSpec((bm, 2), lambda i: (i, 0)),
        ],
        out_shape=[
            jax.ShapeDtypeStruct((nch, N, fdim), jnp.float32),
            jax.ShapeDtypeStruct((N, 2), jnp.float32),
        ],
    )


@functools.lru_cache(maxsize=None)
def _act_kernel(kind, cols):
    bm = 1000

    def body(x_ref, o_ref):
        v = x_ref[...]
        if kind == "mean":
            o_ref[...] = jnp.clip(jnp.exp(v), 1e-5, 1e6)
        elif kind == "disp":
            sp = jnp.maximum(v, 0.0) + jnp.log1p(jnp.exp(-jnp.abs(v)))
            o_ref[...] = jnp.clip(sp, 1e-4, 1e4)
        else:
            o_ref[...] = 1.0 / (1.0 + jnp.exp(-v))

    return pl.pallas_call(
        body,
        grid=(N // bm,),
        in_specs=[pl.BlockSpec((bm, cols), lambda i: (i, 0))],
        out_specs=pl.BlockSpec((bm, cols), lambda i: (i, 0)),
        out_shape=jax.ShapeDtypeStruct((N, cols), jnp.float32),
    )


@functools.lru_cache(maxsize=None)
def _sc_gat(cdim, fdim):
    nch = cdim // fdim
    ncps = nch // NC
    nv = fdim // 16
    mesh = plsc.VectorSubcoreMesh(core_axis_name="c", subcore_axis_name="s")

    @functools.partial(
        pl.kernel,
        mesh=mesh,
        out_type=jax.ShapeDtypeStruct((nch * N, fdim), jnp.float32),
        scratch_types=[
            pltpu.VMEM((N,), jnp.float32),
            pltpu.VMEM((N,), jnp.float32),
            pltpu.VMEM((N,), jnp.float32),
            pltpu.VMEM((N,), jnp.int32),
            pltpu.VMEM((NB, 16), jnp.float32),
            pltpu.VMEM((NB, 16), jnp.int32),
            pltpu.VMEM((NB, 16), jnp.int32),
            pltpu.VMEM((16, fdim), jnp.float32),
            pltpu.VMEM((ZR, fdim), jnp.float32),
            pltpu.VMEM_SHARED((N,), jnp.float32),
            pltpu.VMEM_SHARED((N, fdim), jnp.float32),
            pltpu.SemaphoreType.DMA,
        ],
    )
    def k(xp_hbm, asrc_hbm, adst_hbm, src_hbm, dst_hbm, out_hbm,
          asrc_t, adst_t, denom_t, iota_t, exv, srcv, dstv, rowbuf, zbuf,
          sdenom, sacc, sem):
        cid = lax.axis_index("c")
        sid = lax.axis_index("s")

        pltpu.sync_copy(asrc_hbm, asrc_t)
        pltpu.sync_copy(adst_hbm, adst_t)
        pltpu.sync_copy(src_hbm.at[sid], srcv)
        pltpu.sync_copy(dst_hbm.at[sid], dstv)

        lane = lax.iota(jnp.int32, (16,))

        def init_body(i, carry):
            denom_t[pl.ds(i * 16, 16)] = jnp.zeros((16,), jnp.float32)
            iota_t[pl.ds(i * 16, 16)] = lane + i * 16
            return carry

        lax.fori_loop(0, N // 16, init_body, 0)

        def zb_body(i, carry):
            for v in range(nv):
                zbuf[i, pl.ds(v * 16, 16)] = jnp.zeros((16,), jnp.float32)
            return carry

        lax.fori_loop(0, ZR, zb_body, 0)

        def p1_body(b, carry):
            sv = srcv[b]
            dv = dstv[b]
            a_s = plsc.load_gather(asrc_t, [sv])
            a_d = plsc.load_gather(adst_t, [dv])
            e = 1.0 / (1.0 + jnp.exp(-(a_s + a_d)))
            ex = jnp.exp(e)
            exv[b] = ex
            plsc.addupdate_scatter(denom_t, [dv], ex)
            return carry

        lax.fori_loop(0, NB, p1_body, 0)

        @pl.when(sid == 0)
        def _():
            pltpu.sync_copy(denom_t, sdenom)

        plsc.subcore_barrier()

        @pl.when(sid != 0)
        def _():
            pltpu.sync_copy(denom_t, sdenom.at[iota_t], add=True)

        plsc.subcore_barrier()
        pltpu.sync_copy(sdenom, denom_t)

        def p2_body(b, carry):
            dv = dstv[b]
            d = plsc.load_gather(denom_t, [dv])
            exv[b] = exv[b] / (d + 1e-16)
            return carry

        lax.fori_loop(0, NB, p2_body, 0)

        for i in range(ncps):
            chunk = cid * ncps + i
            base = chunk * N

            for kk in range(RPT // ZR):
                pltpu.sync_copy(
                    zbuf, sacc.at[pl.ds(sid * RPT + kk * ZR, ZR)])
            plsc.subcore_barrier()

            def ce_body(b, carry):
                sv = srcv[b] + base
                pltpu.async_copy(xp_hbm.at[sv], rowbuf, sem).wait()
                for j in range(16):
                    a = exv[b, j]
                    for v in range(nv):
                        rowbuf[j, pl.ds(v * 16, 16)] = (
                            rowbuf[j, pl.ds(v * 16, 16)] * a)
                dv = dstv[b]
                pltpu.sync_copy(rowbuf, sacc.at[dv], add=True)
                return carry

            lax.fori_loop(0, NB, ce_body, 0)
            plsc.subcore_barrier()

            pltpu.sync_copy(
                sacc.at[pl.ds(sid * RPT, RPT)],
                out_hbm.at[pl.ds(base + sid * RPT, RPT)])
            plsc.subcore_barrier()

    return k


def _gat(xin, p, act, src_r, dst_r):
    din, cdim = p["W"].shape
    fdim = 128 if cdim >= 256 else 64
    nch = cdim // fdim
    a2 = jnp.stack([p["a_src"], p["a_dst"]], axis=1)
    xp3, av = _phase_a(din, cdim, fdim, act)(xin, p["W"], a2)
    out = _sc_gat(cdim, fdim)(
        xp3.reshape(nch * N, fdim),
        av[:, 0], av[:, 1], src_r, dst_r)
    return out.reshape(nch, N, fdim).transpose(1, 0, 2).reshape(N, cdim)


def kernel(x, edge_index, params):
    src_r = edge_index[0].reshape(NT, NB, 16)
    dst_r = edge_index[1].reshape(NT, NB, 16)

    def gat(xin, name, act):
        return _gat(xin, params[name], act, src_r, dst_r)

    h1 = gat(x, "conv1", None)
    z = gat(h1, "conv2", "elu")
    hp = gat(x, "psd1", None)
    zp = gat(hp, "psd2", "elu")
    hs = gat(x, "std1", None)
    zs = gat(hs, "std2", "elu")
    zg = 0.5 * (zp + zs)
    h3 = gat(jnp.concatenate([z, zg], axis=1), "conv3", None)
    mean_r = gat(h3, "mean", "elu")
    disp_r = gat(h3, "disp", "elu")
    pi_r = gat(h3, "pi", "elu")
    hg = gat(zg, "gene3", None)
    mg_r = gat(hg, "mean_gene", "elu")
    dg_r = gat(hg, "disp_gene", "elu")

    mean = _act_kernel("mean", 256)(mean_r)
    disp = _act_kernel("disp", 256)(disp_r)
    pi = _act_kernel("pi", 256)(pi_r)
    mg = _act_kernel("mean", 256)(mg_r)
    dg = _act_kernel("disp", 256)(dg_r)
    return jnp.concatenate([mean, disp, pi, mg, dg], axis=1)


# speedup vs baseline: 14.5781x; 14.5781x over previous
import functools

import jax
import jax.numpy as jnp
from jax import lax
from jax.experimental import pallas as pl
from jax.experimental.pallas import tpu as pltpu
from jax.experimental.pallas import tpu_sc as plsc

N = 10000
E = 160000
NT = 16
NC = 2
EPT = E // NT
NB = EPT // 16


@functools.lru_cache(maxsize=None)
def _phase_a(din, cdim, fdim, act):
    nch = cdim // fdim
    bm = 400

    def body(x_ref, w_ref, a2_ref, out_ref, a_ref):
        xb = x_ref[...]
        if act == "elu":
            xb = jnp.where(xb > 0, xb, jnp.exp(xb) - 1.0)
        prod = jnp.dot(xb, w_ref[...], preferred_element_type=jnp.float32)
        for c in range(nch):
            out_ref[c] = prod[:, c * fdim:(c + 1) * fdim]
        a_ref[...] = jnp.dot(prod, a2_ref[...],
                             preferred_element_type=jnp.float32)

    return pl.pallas_call(
        body,
        grid=(N // bm,),
        in_specs=[
            pl.BlockSpec((bm, din), lambda i: (i, 0)),
            pl.BlockSpec((din, cdim), lambda i: (0, 0)),
            pl.BlockSpec((cdim, 2), lambda i: (0, 0)),
        ],
        out_specs=[
            pl.BlockSpec((nch, bm, fdim), lambda i: (0, i, 0)),
            pl.BlockSpec((bm, 2), lambda i: (i, 0)),
        ],
        out_shape=[
            jax.ShapeDtypeStruct((nch, N, fdim), jnp.float32),
            jax.ShapeDtypeStruct((N, 2), jnp.float32),
        ],
    )


@functools.lru_cache(maxsize=None)
def _act_kernel(kind, cols):
    bm = 1000

    def body(x_ref, o_ref):
        v = x_ref[...]
        if kind == "mean":
            o_ref[...] = jnp.clip(jnp.exp(v), 1e-5, 1e6)
        elif kind == "disp":
            sp = jnp.maximum(v, 0.0) + jnp.log(1.0 + jnp.exp(-jnp.abs(v)))
            o_ref[...] = jnp.clip(sp, 1e-4, 1e4)
        else:
            o_ref[...] = 1.0 / (1.0 + jnp.exp(-v))

    return pl.pallas_call(
        body,
        grid=(N // bm,),
        in_specs=[pl.BlockSpec((bm, cols), lambda i: (i, 0))],
        out_specs=pl.BlockSpec((bm, cols), lambda i: (i, 0)),
        out_shape=jax.ShapeDtypeStruct((N, cols), jnp.float32),
    )


HN = N // NC
HP = 5120
CS = HP // NT
FR = 312
ZR = 104


@functools.lru_cache(maxsize=None)
def _sc_gat(cdim):
    fdim = 128
    nch = cdim // fdim
    nv = fdim // 16
    mesh = plsc.VectorSubcoreMesh(core_axis_name="c", subcore_axis_name="s")

    @functools.partial(
        pl.kernel,
        mesh=mesh,
        compiler_params=pltpu.CompilerParams(needs_layout_passes=False),
        out_type=jax.ShapeDtypeStruct((nch * N, fdim), jnp.float32),
        scratch_types=[
            pltpu.VMEM((N,), jnp.float32),
            pltpu.VMEM((N,), jnp.float32),
            pltpu.VMEM((HP,), jnp.float32),
            pltpu.VMEM((1, 128), jnp.int32),
            pltpu.VMEM((EPT + 160,), jnp.float32),
            pltpu.VMEM((EPT + 160,), jnp.int32),
            pltpu.VMEM((EPT + 160,), jnp.int32),
            pltpu.VMEM((16, fdim), jnp.float32),
            pltpu.VMEM((16, fdim), jnp.float32),
            pltpu.VMEM((16, fdim), jnp.float32),
            pltpu.VMEM((16, fdim), jnp.float32),
            pltpu.VMEM((16, fdim), jnp.float32),
            pltpu.VMEM((16, fdim), jnp.float32),
            pltpu.VMEM((ZR, fdim), jnp.float32),
            pltpu.VMEM_SHARED((HP,), jnp.float32),
            pltpu.VMEM_SHARED((HN, fdim), jnp.float32),
            pltpu.SemaphoreType.DMA,
            pltpu.SemaphoreType.DMA,
            pltpu.SemaphoreType.DMA,
            pltpu.SemaphoreType.DMA,
            pltpu.SemaphoreType.DMA,
            pltpu.SemaphoreType.DMA,
        ],
    )
    def k(xp_hbm, asrc_hbm, adst_hbm, src_hbm, dst_hbm, out_hbm,
          asrc_t, adst_t, denom_t, iorow, exv, srcv, dstv,
          gb0, gb1, gb2, gb3, sb0, sb1, zbuf,
          sdenom, sacc, sg0, sg1, sg2, sg3, ss0, ss1):
        cid = lax.axis_index("c")
        sid = lax.axis_index("s")
        nbase = cid * HN

        pltpu.sync_copy(asrc_hbm, asrc_t)
        pltpu.sync_copy(adst_hbm, adst_t)
        pltpu.sync_copy(src_hbm.at[sid], srcv)
        pltpu.sync_copy(dst_hbm.at[sid], dstv)

        lane = lax.iota(jnp.int32, 16)

        def init_body(i, carry):
            denom_t[pl.ds(i * 16, 16)] = jnp.zeros((16,), jnp.float32)
            return carry

        lax.fori_loop(0, HP // 16, init_body, 0)

        def zb_body(i, carry):
            for v in range(nv):
                zbuf[i, pl.ds(v * 16, 16)] = jnp.zeros((16,), jnp.float32)
            return carry

        lax.fori_loop(0, ZR, zb_body, 0)

        def c_body(b, cnt):
            sv = srcv[pl.ds(b * 16, 16)]
            lv = dstv[pl.ds(b * 16, 16)] - nbase
            msk = (lv >= 0) & (lv < HN)
            plsc.store_compressed(srcv.at[pl.ds(cnt, 16)], sv, mask=msk)
            plsc.store_compressed(dstv.at[pl.ds(cnt, 16)], lv, mask=msk)
            return cnt + plsc.all_reduce_population_count(msk)[0]

        cnt = lax.fori_loop(0, NB, c_body, 0)
        for t in range(10):
            srcv[pl.ds(cnt + t * 16, 16)] = jnp.zeros((16,), jnp.int32)
            dstv[pl.ds(cnt + t * 16, 16)] = jnp.full((16,), HN, jnp.int32)
        nb16 = (cnt + 64) // 64 * 4

        def p1_body(b, carry):
            sv = srcv[pl.ds(b * 16, 16)]
            lv = dstv[pl.ds(b * 16, 16)]
            a_s = plsc.load_gather(asrc_t, [sv])
            gd = jnp.where(lv < HN, lv + nbase, 0)
            a_d = plsc.load_gather(adst_t, [gd])
            e = 1.0 / (1.0 + jnp.exp(-(a_s + a_d)))
            ex = jnp.exp(e)
            exv[pl.ds(b * 16, 16)] = ex
            plsc.addupdate_scatter(denom_t, [lv], ex)
            return carry

        lax.fori_loop(0, nb16, p1_body, 0)

        @pl.when(sid == 0)
        def _():
            pltpu.sync_copy(denom_t, sdenom)

        plsc.subcore_barrier()

        @pl.when(sid != 0)
        def _():
            def add_body(j, carry):
                for v in range(8):
                    iorow[0, pl.ds(v * 16, 16)] = lane + (j * 128 + v * 16)
                pltpu.sync_copy(denom_t.at[pl.ds(j * 128, 128)],
                                sdenom.at[iorow.at[0]], add=True)
                return carry

            lax.fori_loop(0, HP // 128, add_body, 0)

        plsc.subcore_barrier()
        pltpu.sync_copy(sdenom, denom_t)

        def p2_body(b, carry):
            lv = dstv[pl.ds(b * 16, 16)]
            d = plsc.load_gather(denom_t, [lv])
            al = exv[pl.ds(b * 16, 16)] / (d + 1e-16)
            exv[pl.ds(b * 16, 16)] = jnp.where(lv < HN, al, 0.0)
            return carry

        lax.fori_loop(0, nb16, p2_body, 0)

        gbufs = (gb0, gb1, gb2, gb3)
        gsems = (sg0, sg1, sg2, sg3)
        sbufs = (sb0, sb1)
        ssems = (ss0, ss1)

        def chunk_body(chunk, carry):
            @pl.when(chunk > 0)
            def _():
                def bump(i, c2):
                    srcv[pl.ds(i * 16, 16)] = srcv[pl.ds(i * 16, 16)] + N
                    return c2

                lax.fori_loop(0, (EPT + 160) // 16, bump, 0)

            for kk in range(FR // ZR):
                pltpu.sync_copy(
                    zbuf, sacc.at[pl.ds(sid * FR + kk * ZR, ZR)])

            @pl.when(sid == NT - 1)
            def _():
                pltpu.sync_copy(zbuf.at[pl.ds(0, 8)],
                                sacc.at[pl.ds(NT * FR, 8)])

            plsc.subcore_barrier()

            def g_desc(b, gb, sg):
                sv = srcv[pl.ds(b * 16, 16)]
                return pltpu.make_async_copy(xp_hbm.at[sv], gb, sg)

            def dvec(b):
                dv = dstv[pl.ds(b * 16, 16)]
                return jnp.where(dv < HN, dv, 0)

            def s_desc(b, sb, ss):
                return pltpu.make_async_copy(sb, sacc.at[dvec(b)], ss)

            def scale(b, gb, sb):
                av = exv[pl.ds(b * 16, 16)]
                for j in range(16):
                    a = av[j]
                    for v in range(nv):
                        sb[j, pl.ds(v * 16, 16)] = (
                            gb[j, pl.ds(v * 16, 16)] * a)

            for u in range(4):
                g_desc(u, gbufs[u], gsems[u]).start()

            def quad(q, carry2):
                for u in range(4):
                    b = q * 4 + u
                    g_desc(b, gbufs[u], gsems[u]).wait()

                    @pl.when(b >= 2)
                    def _():
                        s_desc(b - 2, sbufs[u % 2], ssems[u % 2]).wait()

                    scale(b, gbufs[u], sbufs[u % 2])
                    s_desc(b, sbufs[u % 2], ssems[u % 2]).start(add=True)

                    @pl.when(b + 4 < nb16)
                    def _():
                        g_desc(b + 4, gbufs[u], gsems[u]).start()
                return carry2

            lax.fori_loop(0, nb16 // 4, quad, 0)

            bt = nb16 - 1
            s_desc(bt - 1, sbufs[0], ssems[0]).wait()
            s_desc(bt, sbufs[1], ssems[1]).wait()

            plsc.subcore_barrier()

            pltpu.sync_copy(
                sacc.at[pl.ds(sid * FR, FR)],
                out_hbm.at[pl.ds(chunk * N + nbase + sid * FR, FR)])

            @pl.when(sid == NT - 1)
            def _():
                pltpu.sync_copy(
                    sacc.at[pl.ds(NT * FR, 8)],
                    out_hbm.at[pl.ds(chunk * N + nbase + NT * FR, 8)])

            plsc.subcore_barrier()
            return carry

        lax.fori_loop(0, nch, chunk_body, 0)

    return k


def _gat(xin, p, act, src_r, dst_r):
    din, cdim = p["W"].shape
    fdim = 128
    nch = cdim // fdim
    a2 = jnp.stack([p["a_src"], p["a_dst"]], axis=1)
    xp3, av = _phase_a(din, cdim, fdim, act)(xin, p["W"], a2)
    out = _sc_gat(cdim)(
        xp3.reshape(nch * N, fdim),
        av[:, 0], av[:, 1], src_r, dst_r)
    return out.reshape(nch, N, fdim).transpose(1, 0, 2).reshape(N, cdim)


def kernel(x, edge_index, params):
    src_r = jnp.pad(edge_index[0].reshape(NT, EPT), ((0, 0), (0, 160)))
    dst_r = jnp.pad(edge_index[1].reshape(NT, EPT), ((0, 0), (0, 160)))

    def gat(xin, name, act):
        return _gat(xin, params[name], act, src_r, dst_r)

    h1 = gat(x, "conv1", None)
    z = gat(h1, "conv2", "elu")
    hp = gat(x, "psd1", None)
    zp = gat(hp, "psd2", "elu")
    hs = gat(x, "std1", None)
    zs = gat(hs, "std2", "elu")
    zg = 0.5 * (zp + zs)
    h3 = gat(jnp.concatenate([z, zg], axis=1), "conv3", None)
    mean_r = gat(h3, "mean", "elu")
    disp_r = gat(h3, "disp", "elu")
    pi_r = gat(h3, "pi", "elu")
    hg = gat(zg, "gene3", None)
    mg_r = gat(hg, "mean_gene", "elu")
    dg_r = gat(hg, "disp_gene", "elu")

    mean = _act_kernel("mean", 256)(mean_r)
    disp = _act_kernel("disp", 256)(disp_r)
    pi = _act_kernel("pi", 256)(pi_r)
    mg = _act_kernel("mean", 256)(mg_r)
    dg = _act_kernel("disp", 256)(dg_r)
    return jnp.concatenate([mean, disp, pi, mg, dg], axis=1)


# speedup vs baseline: 14.7578x; 1.0123x over previous
import functools

import jax
import jax.numpy as jnp
from jax import lax
from jax.experimental import pallas as pl
from jax.experimental.pallas import tpu as pltpu
from jax.experimental.pallas import tpu_sc as plsc

N = 10000
E = 160000
NT = 16
NC = 2
EPT = E // NT
NB = EPT // 16


@functools.lru_cache(maxsize=None)
def _phase_a(din, cdim, fdim, act):
    nch = cdim // fdim
    bm = 400

    def body(x_ref, w_ref, a2_ref, out_ref, a_ref):
        xb = x_ref[...]
        if act == "elu":
            xb = jnp.where(xb > 0, xb, jnp.exp(xb) - 1.0)
        prod = jnp.dot(xb, w_ref[...], preferred_element_type=jnp.float32)
        for c in range(nch):
            out_ref[c] = prod[:, c * fdim:(c + 1) * fdim]
        a_ref[...] = jnp.dot(prod, a2_ref[...],
                             preferred_element_type=jnp.float32)

    return pl.pallas_call(
        body,
        grid=(N // bm,),
        in_specs=[
            pl.BlockSpec((bm, din), lambda i: (i, 0)),
            pl.BlockSpec((din, cdim), lambda i: (0, 0)),
            pl.BlockSpec((cdim, 2), lambda i: (0, 0)),
        ],
        out_specs=[
            pl.BlockSpec((nch, bm, fdim), lambda i: (0, i, 0)),
            pl.BlockSpec((bm, 2), lambda i: (i, 0)),
        ],
        out_shape=[
            jax.ShapeDtypeStruct((nch, N, fdim), jnp.float32),
            jax.ShapeDtypeStruct((N, 2), jnp.float32),
        ],
    )


@functools.lru_cache(maxsize=None)
def _act_kernel(kind, cols):
    bm = 1000

    def body(x_ref, o_ref):
        v = x_ref[...]
        if kind == "mean":
            o_ref[...] = jnp.clip(jnp.exp(v), 1e-5, 1e6)
        elif kind == "disp":
            sp = jnp.maximum(v, 0.0) + jnp.log(1.0 + jnp.exp(-jnp.abs(v)))
            o_ref[...] = jnp.clip(sp, 1e-4, 1e4)
        else:
            o_ref[...] = 1.0 / (1.0 + jnp.exp(-v))

    return pl.pallas_call(
        body,
        grid=(N // bm,),
        in_specs=[pl.BlockSpec((bm, cols), lambda i: (i, 0))],
        out_specs=pl.BlockSpec((bm, cols), lambda i: (i, 0)),
        out_shape=jax.ShapeDtypeStruct((N, cols), jnp.float32),
    )


HN = N // NC
HP = 5120
CS = HP // NT
FR = 312
ZR = 104


@functools.lru_cache(maxsize=None)
def _sc_gat(cdim):
    fdim = 128
    nch = cdim // fdim
    nv = fdim // 16
    mesh = plsc.VectorSubcoreMesh(core_axis_name="c", subcore_axis_name="s")

    @functools.partial(
        pl.kernel,
        mesh=mesh,
        compiler_params=pltpu.CompilerParams(needs_layout_passes=False),
        out_type=jax.ShapeDtypeStruct((nch * N, fdim), jnp.float32),
        scratch_types=[
            pltpu.VMEM((N,), jnp.float32),
            pltpu.VMEM((N,), jnp.float32),
            pltpu.VMEM((HP,), jnp.float32),
            pltpu.VMEM((1, 128), jnp.int32),
            pltpu.VMEM((EPT + 160,), jnp.float32),
            pltpu.VMEM((EPT + 160,), jnp.int32),
            pltpu.VMEM((EPT + 160,), jnp.int32),
            pltpu.VMEM((16, fdim), jnp.float32),
            pltpu.VMEM((16, fdim), jnp.float32),
            pltpu.VMEM((16, fdim), jnp.float32),
            pltpu.VMEM((16, fdim), jnp.float32),
            pltpu.VMEM((16, fdim), jnp.float32),
            pltpu.VMEM((16, fdim), jnp.float32),
            pltpu.VMEM((ZR, fdim), jnp.float32),
            pltpu.VMEM_SHARED((HP,), jnp.float32),
            pltpu.VMEM_SHARED((HN, fdim), jnp.float32),
            pltpu.SemaphoreType.DMA,
            pltpu.SemaphoreType.DMA,
            pltpu.SemaphoreType.DMA,
            pltpu.SemaphoreType.DMA,
            pltpu.SemaphoreType.DMA,
            pltpu.SemaphoreType.DMA,
        ],
    )
    def k(xp_hbm, asrc_hbm, adst_hbm, src_hbm, dst_hbm, out_hbm,
          asrc_t, adst_t, denom_t, iorow, exv, srcv, dstv,
          gb0, gb1, gb2, gb3, sb0, sb1, zbuf,
          sdenom, sacc, sg0, sg1, sg2, sg3, ss0, ss1):
        cid = lax.axis_index("c")
        sid = lax.axis_index("s")
        nbase = cid * HN

        pltpu.sync_copy(asrc_hbm, asrc_t)
        pltpu.sync_copy(adst_hbm, adst_t)
        pltpu.sync_copy(src_hbm.at[sid], srcv)
        pltpu.sync_copy(dst_hbm.at[sid], dstv)

        lane = lax.iota(jnp.int32, 16)

        def init_body(i, carry):
            denom_t[pl.ds(i * 16, 16)] = jnp.zeros((16,), jnp.float32)
            return carry

        lax.fori_loop(0, HP // 16, init_body, 0)

        def zb_body(i, carry):
            for v in range(nv):
                zbuf[i, pl.ds(v * 16, 16)] = jnp.zeros((16,), jnp.float32)
            return carry

        lax.fori_loop(0, ZR, zb_body, 0)

        def c_body(b, cnt):
            sv = srcv[pl.ds(b * 16, 16)]
            lv = dstv[pl.ds(b * 16, 16)] - nbase
            msk = (lv >= 0) & (lv < HN)
            plsc.store_compressed(srcv.at[pl.ds(cnt, 16)], sv, mask=msk)
            plsc.store_compressed(dstv.at[pl.ds(cnt, 16)], lv, mask=msk)
            return cnt + plsc.all_reduce_population_count(msk)[0]

        cnt = lax.fori_loop(0, NB, c_body, 0)
        for t in range(10):
            srcv[pl.ds(cnt + t * 16, 16)] = jnp.zeros((16,), jnp.int32)
            dstv[pl.ds(cnt + t * 16, 16)] = jnp.full((16,), HN, jnp.int32)
        nb16 = (cnt + 64) // 64 * 4

        def p1_body(b, carry):
            sv = srcv[pl.ds(b * 16, 16)]
            lv = dstv[pl.ds(b * 16, 16)]
            a_s = plsc.load_gather(asrc_t, [sv])
            gd = jnp.where(lv < HN, lv + nbase, 0)
            a_d = plsc.load_gather(adst_t, [gd])
            e = 1.0 / (1.0 + jnp.exp(-(a_s + a_d)))
            ex = jnp.exp(e)
            exv[pl.ds(b * 16, 16)] = ex
            plsc.addupdate_scatter(denom_t, [lv], ex)
            return carry

        lax.fori_loop(0, nb16, p1_body, 0)

        @pl.when(sid == 0)
        def _():
            pltpu.sync_copy(denom_t, sdenom)

        plsc.subcore_barrier()

        @pl.when(sid != 0)
        def _():
            def add_body(j, carry):
                for v in range(8):
                    iorow[0, pl.ds(v * 16, 16)] = lane + (j * 128 + v * 16)
                pltpu.sync_copy(denom_t.at[pl.ds(j * 128, 128)],
                                sdenom.at[iorow.at[0]], add=True)
                return carry

            lax.fori_loop(0, HP // 128, add_body, 0)

        plsc.subcore_barrier()
        pltpu.sync_copy(sdenom, denom_t)

        def p2_body(b, carry):
            lv = dstv[pl.ds(b * 16, 16)]
            d = plsc.load_gather(denom_t, [lv])
            al = exv[pl.ds(b * 16, 16)] / (d + 1e-16)
            exv[pl.ds(b * 16, 16)] = jnp.where(lv < HN, al, 0.0)
            dstv[pl.ds(b * 16, 16)] = jnp.where(lv < HN, lv, 0)
            return carry

        lax.fori_loop(0, nb16, p2_body, 0)

        gbufs = (gb0, gb1, gb2, gb3)
        gsems = (sg0, sg1, sg2, sg3)
        sbufs = (sb0, sb1)
        ssems = (ss0, ss1)

        def chunk_body(chunk, carry):
            for kk in range(FR // ZR):
                pltpu.sync_copy(
                    zbuf, sacc.at[pl.ds(sid * FR + kk * ZR, ZR)])

            @pl.when(sid == NT - 1)
            def _():
                pltpu.sync_copy(zbuf.at[pl.ds(0, 8)],
                                sacc.at[pl.ds(NT * FR, 8)])

            plsc.subcore_barrier()

            def g_desc(b, gb, sg):
                sv = srcv[pl.ds(b * 16, 16)]
                return pltpu.make_async_copy(xp_hbm.at[chunk].at[sv], gb, sg)

            def s_desc(b, sb, ss):
                dv = dstv[pl.ds(b * 16, 16)]
                return pltpu.make_async_copy(sb, sacc.at[dv], ss)

            def scale(b, gb, sb):
                av = exv[pl.ds(b * 16, 16)]
                for j in range(16):
                    a = av[j]
                    for v in range(nv):
                        sb[j, pl.ds(v * 16, 16)] = (
                            gb[j, pl.ds(v * 16, 16)] * a)

            for u in range(4):
                g_desc(u, gbufs[u], gsems[u]).start()

            def quad(q, carry2):
                for u in range(4):
                    b = q * 4 + u
                    g_desc(b, gbufs[u], gsems[u]).wait()

                    @pl.when(b >= 2)
                    def _():
                        s_desc(b - 2, sbufs[u % 2], ssems[u % 2]).wait()

                    scale(b, gbufs[u], sbufs[u % 2])
                    s_desc(b, sbufs[u % 2], ssems[u % 2]).start(add=True)

                    @pl.when(b + 4 < nb16)
                    def _():
                        g_desc(b + 4, gbufs[u], gsems[u]).start()
                return carry2

            lax.fori_loop(0, nb16 // 4, quad, 0)

            bt = nb16 - 1
            s_desc(bt - 1, sbufs[0], ssems[0]).wait()
            s_desc(bt, sbufs[1], ssems[1]).wait()

            plsc.subcore_barrier()

            pltpu.sync_copy(
                sacc.at[pl.ds(sid * FR, FR)],
                out_hbm.at[pl.ds(chunk * N + nbase + sid * FR, FR)])

            @pl.when(sid == NT - 1)
            def _():
                pltpu.sync_copy(
                    sacc.at[pl.ds(NT * FR, 8)],
                    out_hbm.at[pl.ds(chunk * N + nbase + NT * FR, 8)])

            plsc.subcore_barrier()
            return carry

        lax.fori_loop(0, nch, chunk_body, 0)

    return k


def _gat(xin, p, act, src_r, dst_r):
    din, cdim = p["W"].shape
    fdim = 128
    nch = cdim // fdim
    a2 = jnp.stack([p["a_src"], p["a_dst"]], axis=1)
    xp3, av = _phase_a(din, cdim, fdim, act)(xin, p["W"], a2)
    out = _sc_gat(cdim)(
        xp3, av[:, 0], av[:, 1], src_r, dst_r)
    return out.reshape(nch, N, fdim).transpose(1, 0, 2).reshape(N, cdim)


def kernel(x, edge_index, params):
    src_r = jnp.pad(edge_index[0].reshape(NT, EPT), ((0, 0), (0, 160)))
    dst_r = jnp.pad(edge_index[1].reshape(NT, EPT), ((0, 0), (0, 160)))

    def gat(xin, name, act):
        return _gat(xin, params[name], act, src_r, dst_r)

    h1 = gat(x, "conv1", None)
    z = gat(h1, "conv2", "elu")
    hp = gat(x, "psd1", None)
    zp = gat(hp, "psd2", "elu")
    hs = gat(x, "std1", None)
    zs = gat(hs, "std2", "elu")
    zg = 0.5 * (zp + zs)
    h3 = gat(jnp.concatenate([z, zg], axis=1), "conv3", None)
    mean_r = gat(h3, "mean", "elu")
    disp_r = gat(h3, "disp", "elu")
    pi_r = gat(h3, "pi", "elu")
    hg = gat(zg, "gene3", None)
    mg_r = gat(hg, "mean_gene", "elu")
    dg_r = gat(hg, "disp_gene", "elu")

    mean = _act_kernel("mean", 256)(mean_r)
    disp = _act_kernel("disp", 256)(disp_r)
    pi = _act_kernel("pi", 256)(pi_r)
    mg = _act_kernel("mean", 256)(mg_r)
    dg = _act_kernel("disp", 256)(dg_r)
    return jnp.concatenate([mean, disp, pi, mg, dg], axis=1)


# speedup vs baseline: 14.8604x; 1.0070x over previous
import functools

import jax
import jax.numpy as jnp
from jax import lax
from jax.experimental import pallas as pl
from jax.experimental.pallas import tpu as pltpu
from jax.experimental.pallas import tpu_sc as plsc

N = 10000
E = 160000
NT = 16
NC = 2
EPT = E // NT
NB = EPT // 16


@functools.lru_cache(maxsize=None)
def _phase_a(din, cdim, fdim, act):
    nch = cdim // fdim
    bm = 400

    def body(x_ref, w_ref, a2_ref, out_ref, a_ref):
        xb = x_ref[...]
        if act == "elu":
            xb = jnp.where(xb > 0, xb, jnp.exp(xb) - 1.0)
        prod = jnp.dot(xb, w_ref[...], preferred_element_type=jnp.float32)
        for c in range(nch):
            out_ref[c] = prod[:, c * fdim:(c + 1) * fdim]
        a_ref[...] = jnp.dot(prod, a2_ref[...],
                             preferred_element_type=jnp.float32)

    return pl.pallas_call(
        body,
        grid=(N // bm,),
        in_specs=[
            pl.BlockSpec((bm, din), lambda i: (i, 0)),
            pl.BlockSpec((din, cdim), lambda i: (0, 0)),
            pl.BlockSpec((cdim, 2), lambda i: (0, 0)),
        ],
        out_specs=[
            pl.BlockSpec((nch, bm, fdim), lambda i: (0, i, 0)),
            pl.BlockSpec((bm, 2), lambda i: (i, 0)),
        ],
        out_shape=[
            jax.ShapeDtypeStruct((nch, N, fdim), jnp.float32),
            jax.ShapeDtypeStruct((N, 2), jnp.float32),
        ],
    )


@functools.lru_cache(maxsize=None)
def _act_kernel(kind, cols):
    bm = 1000

    def body(x_ref, o_ref):
        v = x_ref[...]
        if kind == "mean":
            o_ref[...] = jnp.clip(jnp.exp(v), 1e-5, 1e6)
        elif kind == "disp":
            sp = jnp.maximum(v, 0.0) + jnp.log(1.0 + jnp.exp(-jnp.abs(v)))
            o_ref[...] = jnp.clip(sp, 1e-4, 1e4)
        else:
            o_ref[...] = 1.0 / (1.0 + jnp.exp(-v))

    return pl.pallas_call(
        body,
        grid=(N // bm,),
        in_specs=[pl.BlockSpec((bm, cols), lambda i: (i, 0))],
        out_specs=pl.BlockSpec((bm, cols), lambda i: (i, 0)),
        out_shape=jax.ShapeDtypeStruct((N, cols), jnp.float32),
    )


HN = N // NC
HP = 5120
CS = HP // NT
FR = 312
ZR = 104


@functools.lru_cache(maxsize=None)
def _sc_gat(cdim):
    fdim = 128
    nch = cdim // fdim
    nv = fdim // 16
    mesh = plsc.VectorSubcoreMesh(core_axis_name="c", subcore_axis_name="s")

    @functools.partial(
        pl.kernel,
        mesh=mesh,
        compiler_params=pltpu.CompilerParams(needs_layout_passes=False),
        out_type=jax.ShapeDtypeStruct((nch * N, fdim), jnp.float32),
        scratch_types=[
            pltpu.VMEM((N,), jnp.float32),
            pltpu.VMEM((N,), jnp.float32),
            pltpu.VMEM((HP,), jnp.float32),
            pltpu.VMEM((HP // 128, 128), jnp.int32),
            pltpu.VMEM((EPT + 160,), jnp.float32),
            pltpu.VMEM((EPT + 160,), jnp.int32),
            pltpu.VMEM((EPT + 160,), jnp.int32),
            pltpu.VMEM((16, fdim), jnp.float32),
            pltpu.VMEM((16, fdim), jnp.float32),
            pltpu.VMEM((16, fdim), jnp.float32),
            pltpu.VMEM((16, fdim), jnp.float32),
            pltpu.VMEM((16, fdim), jnp.float32),
            pltpu.VMEM((16, fdim), jnp.float32),
            pltpu.VMEM((ZR, fdim), jnp.float32),
            pltpu.VMEM_SHARED((HP,), jnp.float32),
            pltpu.VMEM_SHARED((HN, fdim), jnp.float32),
            pltpu.SemaphoreType.DMA,
            pltpu.SemaphoreType.DMA,
            pltpu.SemaphoreType.DMA,
            pltpu.SemaphoreType.DMA,
            pltpu.SemaphoreType.DMA,
            pltpu.SemaphoreType.DMA,
        ],
    )
    def k(xp_hbm, asrc_hbm, adst_hbm, src_hbm, dst_hbm, out_hbm,
          asrc_t, adst_t, denom_t, iota2, exv, srcv, dstv,
          gb0, gb1, gb2, gb3, sb0, sb1, zbuf,
          sdenom, sacc, sg0, sg1, sg2, sg3, ss0, ss1):
        cid = lax.axis_index("c")
        sid = lax.axis_index("s")
        nbase = cid * HN

        pltpu.sync_copy(asrc_hbm, asrc_t)
        pltpu.sync_copy(adst_hbm, adst_t)
        pltpu.sync_copy(src_hbm.at[sid], srcv)
        pltpu.sync_copy(dst_hbm.at[sid], dstv)

        lane = lax.iota(jnp.int32, 16)

        def init_body(i, carry):
            denom_t[pl.ds(i * 16, 16)] = jnp.zeros((16,), jnp.float32)
            return carry

        lax.fori_loop(0, HP // 16, init_body, 0)

        def iota_body(j, carry):
            for v in range(8):
                iota2[j, pl.ds(v * 16, 16)] = lane + (j * 128 + v * 16)
            return carry

        lax.fori_loop(0, HP // 128, iota_body, 0)

        def zb_body(i, carry):
            for v in range(nv):
                zbuf[i, pl.ds(v * 16, 16)] = jnp.zeros((16,), jnp.float32)
            return carry

        lax.fori_loop(0, ZR, zb_body, 0)

        def c_body(b, cnt):
            sv = srcv[pl.ds(b * 16, 16)]
            lv = dstv[pl.ds(b * 16, 16)] - nbase
            msk = (lv >= 0) & (lv < HN)
            plsc.store_compressed(srcv.at[pl.ds(cnt, 16)], sv, mask=msk)
            plsc.store_compressed(dstv.at[pl.ds(cnt, 16)], lv, mask=msk)
            return cnt + plsc.all_reduce_population_count(msk)[0]

        cnt = lax.fori_loop(0, NB, c_body, 0)
        for t in range(10):
            srcv[pl.ds(cnt + t * 16, 16)] = jnp.zeros((16,), jnp.int32)
            dstv[pl.ds(cnt + t * 16, 16)] = jnp.full((16,), HN, jnp.int32)
        nb16 = (cnt + 64) // 64 * 4

        def p1_body(b, carry):
            sv = srcv[pl.ds(b * 16, 16)]
            lv = dstv[pl.ds(b * 16, 16)]
            a_s = plsc.load_gather(asrc_t, [sv])
            gd = jnp.where(lv < HN, lv + nbase, 0)
            a_d = plsc.load_gather(adst_t, [gd])
            e = 1.0 / (1.0 + jnp.exp(-(a_s + a_d)))
            ex = jnp.exp(e)
            exv[pl.ds(b * 16, 16)] = ex
            plsc.addupdate_scatter(denom_t, [lv], ex)
            return carry

        lax.fori_loop(0, nb16, p1_body, 0)

        @pl.when(sid == 0)
        def _():
            pltpu.sync_copy(denom_t, sdenom)

        plsc.subcore_barrier()

        @pl.when(sid != 0)
        def _():
            def add_fire(j, carry):
                pltpu.async_copy(denom_t.at[pl.ds(j * 128, 128)],
                                 sdenom.at[iota2.at[j]], sg0, add=True)
                return carry

            lax.fori_loop(0, HP // 128, add_fire, 0)

            def add_drain(j, carry):
                pltpu.make_async_copy(denom_t.at[pl.ds(j * 128, 128)],
                                      sdenom.at[iota2.at[j]], sg0).wait()
                return carry

            lax.fori_loop(0, HP // 128, add_drain, 0)

        plsc.subcore_barrier()
        pltpu.sync_copy(sdenom, denom_t)

        def p2_body(b, carry):
            lv = dstv[pl.ds(b * 16, 16)]
            d = plsc.load_gather(denom_t, [lv])
            al = exv[pl.ds(b * 16, 16)] / (d + 1e-16)
            exv[pl.ds(b * 16, 16)] = jnp.where(lv < HN, al, 0.0)
            dstv[pl.ds(b * 16, 16)] = jnp.where(lv < HN, lv, 0)
            return carry

        lax.fori_loop(0, nb16, p2_body, 0)

        gbufs = (gb0, gb1, gb2, gb3)
        gsems = (sg0, sg1, sg2, sg3)
        sbufs = (sb0, sb1)
        ssems = (ss0, ss1)

        def chunk_body(chunk, carry):
            for kk in range(FR // ZR):
                pltpu.sync_copy(
                    zbuf, sacc.at[pl.ds(sid * FR + kk * ZR, ZR)])

            @pl.when(sid == NT - 1)
            def _():
                pltpu.sync_copy(zbuf.at[pl.ds(0, 8)],
                                sacc.at[pl.ds(NT * FR, 8)])

            plsc.subcore_barrier()

            def g_desc(b, gb, sg):
                sv = srcv[pl.ds(b * 16, 16)]
                return pltpu.make_async_copy(xp_hbm.at[chunk].at[sv], gb, sg)

            def s_desc(b, sb, ss):
                dv = dstv[pl.ds(b * 16, 16)]
                return pltpu.make_async_copy(sb, sacc.at[dv], ss)

            def scale(b, gb, sb):
                av = exv[pl.ds(b * 16, 16)]
                for j in range(16):
                    a = av[j]
                    for v in range(nv):
                        sb[j, pl.ds(v * 16, 16)] = (
                            gb[j, pl.ds(v * 16, 16)] * a)

            for u in range(4):
                g_desc(u, gbufs[u], gsems[u]).start()

            def quad(q, carry2):
                for u in range(4):
                    b = q * 4 + u
                    g_desc(b, gbufs[u], gsems[u]).wait()

                    @pl.when(b >= 2)
                    def _():
                        s_desc(b - 2, sbufs[u % 2], ssems[u % 2]).wait()

                    scale(b, gbufs[u], sbufs[u % 2])
                    s_desc(b, sbufs[u % 2], ssems[u % 2]).start(add=True)

                    @pl.when(b + 4 < nb16)
                    def _():
                        g_desc(b + 4, gbufs[u], gsems[u]).start()
                return carry2

            lax.fori_loop(0, nb16 // 4, quad, 0)

            bt = nb16 - 1
            s_desc(bt - 1, sbufs[0], ssems[0]).wait()
            s_desc(bt, sbufs[1], ssems[1]).wait()

            plsc.subcore_barrier()

            pltpu.sync_copy(
                sacc.at[pl.ds(sid * FR, FR)],
                out_hbm.at[pl.ds(chunk * N + nbase + sid * FR, FR)])

            @pl.when(sid == NT - 1)
            def _():
                pltpu.sync_copy(
                    sacc.at[pl.ds(NT * FR, 8)],
                    out_hbm.at[pl.ds(chunk * N + nbase + NT * FR, 8)])

            plsc.subcore_barrier()
            return carry

        lax.fori_loop(0, nch, chunk_body, 0)

    return k


def _gat(xin, p, act, src_r, dst_r):
    din, cdim = p["W"].shape
    fdim = 128
    nch = cdim // fdim
    a2 = jnp.stack([p["a_src"], p["a_dst"]], axis=1)
    xp3, av = _phase_a(din, cdim, fdim, act)(xin, p["W"], a2)
    out = _sc_gat(cdim)(
        xp3, av[:, 0], av[:, 1], src_r, dst_r)
    return out.reshape(nch, N, fdim).transpose(1, 0, 2).reshape(N, cdim)


def kernel(x, edge_index, params):
    src_r = jnp.pad(edge_index[0].reshape(NT, EPT), ((0, 0), (0, 160)))
    dst_r = jnp.pad(edge_index[1].reshape(NT, EPT), ((0, 0), (0, 160)))

    def gat(xin, name, act):
        return _gat(xin, params[name], act, src_r, dst_r)

    h1 = gat(x, "conv1", None)
    z = gat(h1, "conv2", "elu")
    hp = gat(x, "psd1", None)
    zp = gat(hp, "psd2", "elu")
    hs = gat(x, "std1", None)
    zs = gat(hs, "std2", "elu")
    zg = 0.5 * (zp + zs)
    h3 = gat(jnp.concatenate([z, zg], axis=1), "conv3", None)
    mean_r = gat(h3, "mean", "elu")
    disp_r = gat(h3, "disp", "elu")
    pi_r = gat(h3, "pi", "elu")
    hg = gat(zg, "gene3", None)
    mg_r = gat(hg, "mean_gene", "elu")
    dg_r = gat(hg, "disp_gene", "elu")

    mean = _act_kernel("mean", 256)(mean_r)
    disp = _act_kernel("disp", 256)(disp_r)
    pi = _act_kernel("pi", 256)(pi_r)
    mg = _act_kernel("mean", 256)(mg_r)
    dg = _act_kernel("disp", 256)(dg_r)
    return jnp.concatenate([mean, disp, pi, mg, dg], axis=1)


# speedup vs baseline: 14.8653x; 1.0003x over previous
import functools

import jax
import jax.numpy as jnp
from jax import lax
from jax.experimental import pallas as pl
from jax.experimental.pallas import tpu as pltpu
from jax.experimental.pallas import tpu_sc as plsc

N = 10000
E = 160000
NT = 16
NC = 2
EPT = E // NT
NB = EPT // 16


@functools.lru_cache(maxsize=None)
def _phase_a(din, cdim, fdim, act):
    nch = cdim // fdim
    bm = 400

    def body(x_ref, w_ref, a2_ref, out_ref, a_ref):
        xb = x_ref[...]
        if act == "elu":
            xb = jnp.where(xb > 0, xb, jnp.exp(xb) - 1.0)
        prod = jnp.dot(xb, w_ref[...], preferred_element_type=jnp.float32)
        for c in range(nch):
            out_ref[c] = prod[:, c * fdim:(c + 1) * fdim]
        a_ref[...] = jnp.dot(prod, a2_ref[...],
                             preferred_element_type=jnp.float32)

    return pl.pallas_call(
        body,
        grid=(N // bm,),
        in_specs=[
            pl.BlockSpec((bm, din), lambda i: (i, 0)),
            pl.BlockSpec((din, cdim), lambda i: (0, 0)),
            pl.BlockSpec((cdim, 2), lambda i: (0, 0)),
        ],
        out_specs=[
            pl.BlockSpec((nch, bm, fdim), lambda i: (0, i, 0)),
            pl.BlockSpec((bm, 2), lambda i: (i, 0)),
        ],
        out_shape=[
            jax.ShapeDtypeStruct((nch, N, fdim), jnp.float32),
            jax.ShapeDtypeStruct((N, 2), jnp.float32),
        ],
    )


@functools.lru_cache(maxsize=None)
def _act_kernel(kind, cols):
    bm = 1000

    def body(x_ref, o_ref):
        v = x_ref[...]
        if kind == "mean":
            o_ref[...] = jnp.clip(jnp.exp(v), 1e-5, 1e6)
        elif kind == "disp":
            sp = jnp.maximum(v, 0.0) + jnp.log(1.0 + jnp.exp(-jnp.abs(v)))
            o_ref[...] = jnp.clip(sp, 1e-4, 1e4)
        else:
            o_ref[...] = 1.0 / (1.0 + jnp.exp(-v))

    return pl.pallas_call(
        body,
        grid=(N // bm,),
        in_specs=[pl.BlockSpec((bm, cols), lambda i: (i, 0))],
        out_specs=pl.BlockSpec((bm, cols), lambda i: (i, 0)),
        out_shape=jax.ShapeDtypeStruct((N, cols), jnp.float32),
    )


HN = N // NC
HP = 5120
FR = 312
ZR = 104


@functools.lru_cache(maxsize=None)
def _sc_gat(cdim):
    fdim = 128
    nch = cdim // fdim
    nv = fdim // 16
    mesh = plsc.VectorSubcoreMesh(core_axis_name="c", subcore_axis_name="s")

    @functools.partial(
        pl.kernel,
        mesh=mesh,
        compiler_params=pltpu.CompilerParams(needs_layout_passes=False),
        out_type=jax.ShapeDtypeStruct((nch * N, fdim), jnp.float32),
        scratch_types=[
            pltpu.VMEM((N,), jnp.float32),
            pltpu.VMEM((N,), jnp.float32),
            pltpu.VMEM((HP,), jnp.float32),
            pltpu.VMEM((HP // 128, 128), jnp.int32),
            pltpu.VMEM((EPT + 160,), jnp.float32),
            pltpu.VMEM((EPT + 160,), jnp.int32),
            pltpu.VMEM((EPT + 160,), jnp.int32),
            pltpu.VMEM((16, fdim), jnp.float32),
            pltpu.VMEM((16, fdim), jnp.float32),
            pltpu.VMEM((16, fdim), jnp.float32),
            pltpu.VMEM((16, fdim), jnp.float32),
            pltpu.VMEM((16, fdim), jnp.float32),
            pltpu.VMEM((16, fdim), jnp.float32),
            pltpu.VMEM((ZR, fdim), jnp.float32),
            pltpu.VMEM_SHARED((HP,), jnp.float32),
            pltpu.VMEM_SHARED((HN, fdim), jnp.float32),
            pltpu.SemaphoreType.DMA,
            pltpu.SemaphoreType.DMA,
            pltpu.SemaphoreType.DMA,
            pltpu.SemaphoreType.DMA,
            pltpu.SemaphoreType.DMA,
            pltpu.SemaphoreType.DMA,
        ],
    )
    def k(xp_hbm, asrc_hbm, adst_hbm, src_hbm, dst_hbm, out_hbm,
          asrc_t, adst_t, denom_t, iota2, exv, srcv, dstv,
          gb0, gb1, gb2, gb3, sb0, sb1, zbuf,
          sdenom, sacc, sg0, sg1, sg2, sg3, ss0, ss1):
        cid = lax.axis_index("c")
        sid = lax.axis_index("s")
        nbase = cid * HN

        pltpu.sync_copy(asrc_hbm, asrc_t)
        pltpu.sync_copy(adst_hbm, adst_t)
        pltpu.sync_copy(src_hbm.at[sid], srcv)
        pltpu.sync_copy(dst_hbm.at[sid], dstv)

        lane = lax.iota(jnp.int32, 16)

        def init_body(i, carry):
            denom_t[pl.ds(i * 16, 16)] = jnp.zeros((16,), jnp.float32)
            return carry

        lax.fori_loop(0, HP // 16, init_body, 0)

        def iota_body(j, carry):
            for v in range(8):
                iota2[j, pl.ds(v * 16, 16)] = lane + (j * 128 + v * 16)
            return carry

        lax.fori_loop(0, HP // 128, iota_body, 0)

        def zb_body(i, carry):
            for v in range(nv):
                zbuf[i, pl.ds(v * 16, 16)] = jnp.zeros((16,), jnp.float32)
            return carry

        lax.fori_loop(0, ZR, zb_body, 0)

        def c_body(b, cnt):
            sv = srcv[pl.ds(b * 16, 16)]
            lv = dstv[pl.ds(b * 16, 16)] - nbase
            msk = (lv >= 0) & (lv < HN)
            plsc.store_compressed(srcv.at[pl.ds(cnt, 16)], sv, mask=msk)
            plsc.store_compressed(dstv.at[pl.ds(cnt, 16)], lv, mask=msk)
            return cnt + plsc.all_reduce_population_count(msk)[0]

        cnt = lax.fori_loop(0, NB, c_body, 0)
        for t in range(10):
            srcv[pl.ds(cnt + t * 16, 16)] = jnp.zeros((16,), jnp.int32)
            dstv[pl.ds(cnt + t * 16, 16)] = jnp.full((16,), HN, jnp.int32)
        nb16 = (cnt + 64) // 64 * 4

        def p1_body(b, carry):
            sv = srcv[pl.ds(b * 16, 16)]
            lv = dstv[pl.ds(b * 16, 16)]
            a_s = plsc.load_gather(asrc_t, [sv])
            gd = jnp.where(lv < HN, lv + nbase, 0)
            a_d = plsc.load_gather(adst_t, [gd])
            e = 1.0 / (1.0 + jnp.exp(-(a_s + a_d)))
            ex = jnp.exp(e)
            exv[pl.ds(b * 16, 16)] = ex
            plsc.addupdate_scatter(denom_t, [lv], ex)
            return carry

        lax.fori_loop(0, nb16, p1_body, 0)

        @pl.when(sid == 0)
        def _():
            pltpu.sync_copy(denom_t, sdenom)

        plsc.subcore_barrier()

        @pl.when(sid != 0)
        def _():
            def add_fire(j, carry):
                pltpu.async_copy(denom_t.at[pl.ds(j * 128, 128)],
                                 sdenom.at[iota2.at[j]], sg0, add=True)
                return carry

            lax.fori_loop(0, HP // 128, add_fire, 0)

            def add_drain(j, carry):
                pltpu.make_async_copy(denom_t.at[pl.ds(j * 128, 128)],
                                      sdenom.at[iota2.at[j]], sg0).wait()
                return carry

            lax.fori_loop(0, HP // 128, add_drain, 0)

        plsc.subcore_barrier()
        pltpu.sync_copy(sdenom, denom_t)

        def p2_body(b, carry):
            lv = dstv[pl.ds(b * 16, 16)]
            d = plsc.load_gather(denom_t, [lv])
            al = exv[pl.ds(b * 16, 16)] / (d + 1e-16)
            exv[pl.ds(b * 16, 16)] = jnp.where(lv < HN, al, 0.0)
            dstv[pl.ds(b * 16, 16)] = jnp.where(lv < HN, lv, 0)
            return carry

        lax.fori_loop(0, nb16, p2_body, 0)

        gbufs = (gb0, gb1, gb2, gb3)
        gsems = (sg0, sg1, sg2, sg3)
        sbufs = (sb0, sb1)
        ssems = (ss0, ss1)

        def chunk_body(chunk, carry):
            for kk in range(FR // ZR):
                pltpu.sync_copy(
                    zbuf, sacc.at[pl.ds(sid * FR + kk * ZR, ZR)])

            @pl.when(sid == NT - 1)
            def _():
                pltpu.sync_copy(zbuf.at[pl.ds(0, 8)],
                                sacc.at[pl.ds(NT * FR, 8)])

            plsc.subcore_barrier()

            def g_desc(b, gb, sg):
                sv = srcv[pl.ds(b * 16, 16)]
                return pltpu.make_async_copy(xp_hbm.at[chunk].at[sv], gb, sg)

            def s_desc(b, sb, ss):
                dv = dstv[pl.ds(b * 16, 16)]
                return pltpu.make_async_copy(sb, sacc.at[dv], ss)

            def scale(b, gb, sb):
                av = exv[pl.ds(b * 16, 16)]
                for j in range(16):
                    a = av[j]
                    for v in range(nv):
                        sb[j, pl.ds(v * 16, 16)] = (
                            gb[j, pl.ds(v * 16, 16)] * a)

            for u in range(4):
                g_desc(u, gbufs[u], gsems[u]).start()

            def quad(q, carry2):
                for u in range(4):
                    b = q * 4 + u
                    g_desc(b, gbufs[u], gsems[u]).wait()

                    @pl.when(b >= 2)
                    def _():
                        s_desc(b - 2, sbufs[u % 2], ssems[u % 2]).wait()

                    scale(b, gbufs[u], sbufs[u % 2])
                    s_desc(b, sbufs[u % 2], ssems[u % 2]).start(add=True)

                    @pl.when(b + 4 < nb16)
                    def _():
                        g_desc(b + 4, gbufs[u], gsems[u]).start()
                return carry2

            lax.fori_loop(0, nb16 // 4, quad, 0)

            bt = nb16 - 1
            s_desc(bt - 1, sbufs[0], ssems[0]).wait()
            s_desc(bt, sbufs[1], ssems[1]).wait()

            plsc.subcore_barrier()

            pltpu.sync_copy(
                sacc.at[pl.ds(sid * FR, FR)],
                out_hbm.at[pl.ds(chunk * N + nbase + sid * FR, FR)])

            @pl.when(sid == NT - 1)
            def _():
                pltpu.sync_copy(
                    sacc.at[pl.ds(NT * FR, 8)],
                    out_hbm.at[pl.ds(chunk * N + nbase + NT * FR, 8)])

            plsc.subcore_barrier()
            return carry

        lax.fori_loop(0, nch, chunk_body, 0)

    return k


def _gat(xin, p, act, src_r, dst_r):
    din, cdim = p["W"].shape
    fdim = 128
    nch = cdim // fdim
    a2 = jnp.stack([p["a_src"], p["a_dst"]], axis=1)
    xp3, av = _phase_a(din, cdim, fdim, act)(xin, p["W"], a2)
    out = _sc_gat(cdim)(
        xp3, av[:, 0], av[:, 1], src_r, dst_r)
    return out.reshape(nch, N, fdim).transpose(1, 0, 2).reshape(N, cdim)


def kernel(x, edge_index, params):
    src_r = jnp.pad(edge_index[0].reshape(NT, EPT), ((0, 0), (0, 160)))
    dst_r = jnp.pad(edge_index[1].reshape(NT, EPT), ((0, 0), (0, 160)))

    def gat(xin, name, act):
        return _gat(xin, params[name], act, src_r, dst_r)

    h1 = gat(x, "conv1", None)
    z = gat(h1, "conv2", "elu")
    hp = gat(x, "psd1", None)
    zp = gat(hp, "psd2", "elu")
    hs = gat(x, "std1", None)
    zs = gat(hs, "std2", "elu")
    zg = 0.5 * (zp + zs)
    h3 = gat(jnp.concatenate([z, zg], axis=1), "conv3", None)
    mean_r = gat(h3, "mean", "elu")
    disp_r = gat(h3, "disp", "elu")
    pi_r = gat(h3, "pi", "elu")
    hg = gat(zg, "gene3", None)
    mg_r = gat(hg, "mean_gene", "elu")
    dg_r = gat(hg, "disp_gene", "elu")

    mean = _act_kernel("mean", 256)(mean_r)
    disp = _act_kernel("disp", 256)(disp_r)
    pi = _act_kernel("pi", 256)(pi_r)
    mg = _act_kernel("mean", 256)(mg_r)
    dg = _act_kernel("disp", 256)(dg_r)
    return jnp.concatenate([mean, disp, pi, mg, dg], axis=1)
